# Initial kernel scaffold; baseline (speedup 1.0000x reference)
import jax
import jax.numpy as jnp
from jax.experimental import pallas as pl


def kernel(x_s, edge_index_s, edge_attr_s, batch_s, x_t, edge_index_t, edge_attr_t, batch_t, psi1_W_self, psi1_W_nbr, psi1_W_edge, psi1_b, psi2_W_self, psi2_W_nbr, psi2_W_edge, psi2_b, mlp_W1, mlp_b1, mlp_W2, mlp_b2):
    raise NotImplementedError("write your pallas kernel here")


# speedup vs baseline: 1.7447x; 1.7447x over previous
import functools

import jax
import jax.numpy as jnp
from jax.experimental import pallas as pl
from jax.experimental.pallas import tpu as pltpu

N_S = 10000
N_T = 10000
E = 160000
F_IN = 128
R = 32
D_EDGE = 4
K = 10
NUM_STEPS = 2

_BK = 400


def _knn_body(hs_ref, ht_ref, idx_ref, val_ref, s0_ref):
    hs = hs_ref[...]
    ht = ht_ref[...]
    sim = jax.lax.dot_general(hs, ht, (((1,), (1,)), ((), ())),
                              preferred_element_type=jnp.float32)
    sim_hi = jax.lax.dot_general(hs, ht, (((1,), (1,)), ((), ())),
                                 preferred_element_type=jnp.float32,
                                 precision=jax.lax.Precision.HIGHEST)
    cols = jax.lax.broadcasted_iota(jnp.int32, sim.shape, 1)
    vals, idxs = [], []
    for _ in range(K):
        m = jnp.max(sim, axis=1, keepdims=True)
        i = jnp.min(jnp.where(sim == m, cols, N_T), axis=1, keepdims=True)
        sel = cols == i
        v = jnp.max(jnp.where(sel, sim_hi, -jnp.inf), axis=1, keepdims=True)
        vals.append(v)
        idxs.append(i)
        sim = jnp.where(sel, -jnp.inf, sim)
    v = jnp.concatenate(vals, axis=1)
    idx_ref[...] = jnp.concatenate(idxs, axis=1)
    val_ref[...] = v
    e = jnp.exp(v - v[:, 0:1])
    s0_ref[...] = e / jnp.sum(e, axis=1, keepdims=True)


@jax.jit
def _knn_topk(h_s, h_t):
    grid = (N_S // _BK,)
    return pl.pallas_call(
        _knn_body,
        grid=grid,
        in_specs=[
            pl.BlockSpec((_BK, F_IN), lambda i: (i, 0)),
            pl.BlockSpec((N_T, F_IN), lambda i: (0, 0)),
        ],
        out_specs=[
            pl.BlockSpec((_BK, K), lambda i: (i, 0)),
            pl.BlockSpec((_BK, K), lambda i: (i, 0)),
            pl.BlockSpec((_BK, K), lambda i: (i, 0)),
        ],
        out_shape=[
            jax.ShapeDtypeStruct((N_S, K), jnp.int32),
            jax.ShapeDtypeStruct((N_S, K), jnp.float32),
            jax.ShapeDtypeStruct((N_S, K), jnp.float32),
        ],
    )(h_s, h_t)


def _gnn_layer(x, edge_index, edge_attr, W_self, W_nbr, W_edge, b):
    src, dst = edge_index[0], edge_index[1]
    msg = x[src] @ W_nbr + edge_attr @ W_edge
    agg = jax.ops.segment_sum(msg, dst, num_segments=x.shape[0])
    return jax.nn.relu(x @ W_self + agg + b)


def kernel(x_s, edge_index_s, edge_attr_s, batch_s, x_t, edge_index_t,
           edge_attr_t, batch_t, psi1_W_self, psi1_W_nbr, psi1_W_edge, psi1_b,
           psi2_W_self, psi2_W_nbr, psi2_W_edge, psi2_b, mlp_W1, mlp_b1,
           mlp_W2, mlp_b2):
    h_s = _gnn_layer(x_s, edge_index_s, edge_attr_s,
                     psi1_W_self, psi1_W_nbr, psi1_W_edge, psi1_b)
    h_t = _gnn_layer(x_t, edge_index_t, edge_attr_t,
                     psi1_W_self, psi1_W_nbr, psi1_W_edge, psi1_b)
    S_idx, S_hat, S_0 = _knn_topk(h_s, h_t)
    rkey = jax.random.key(1234)
    for step in range(NUM_STEPS):
        S = jax.nn.softmax(S_hat, axis=-1)
        r_s = jax.random.normal(jax.random.fold_in(rkey, step), (N_S, R),
                                dtype=jnp.float32)
        tmp = (r_s[:, None, :] * S[:, :, None]).reshape(N_S * K, R)
        r_t = jax.ops.segment_sum(tmp, S_idx.reshape(-1), num_segments=N_T)
        o_s = _gnn_layer(r_s, edge_index_s, edge_attr_s,
                         psi2_W_self, psi2_W_nbr, psi2_W_edge, psi2_b)
        o_t = _gnn_layer(r_t, edge_index_t, edge_attr_t,
                         psi2_W_self, psi2_W_nbr, psi2_W_edge, psi2_b)
        D = o_s[:, None, :] - o_t[S_idx]
        hmid = jax.nn.relu(D @ mlp_W1 + mlp_b1)
        upd = (hmid @ mlp_W2 + mlp_b2)[..., 0]
        S_hat = S_hat + upd
    S_L = jax.nn.softmax(S_hat, axis=-1)
    return (S_0[None], S_L[None], S_idx[None])
